# baseline (speedup 1.0000x reference)
import functools

import jax
import jax.numpy as jnp
from jax.experimental import pallas as pl
from jax.experimental.pallas import tpu as pltpu

_BW = 128
_TM = 8192


def _sublane_sum(a):
    s = a.shape[0]
    while s > 1:
        h = s // 2
        a = a[:h] + a[h:s]
        s = h
    return a


def _fused_kernel(scal_ref, x_ref, w1_ref, b1_ref, w2_ref, b2_ref,
                  rcombt_ref, biasm_ref, mask_ref, p_ref,
                  mu_ref, std_ref, *, n_out, nb):
    x = x_ref[0]
    h1 = jnp.maximum(w1_ref[...] * x + b1_ref[...], 0.0)
    h2 = jnp.maximum(
        jnp.dot(w2_ref[...], h1.astype(jnp.bfloat16),
                preferred_element_type=jnp.float32) + b2_ref[...], 0.0)

    pv = p_ref[...]
    onehot = jnp.where((x >= pv[0:nb]) & (x < pv[1:nb + 1]),
                       1.0, 0.0).astype(jnp.bfloat16)

    lhs = jnp.concatenate([h2.astype(jnp.bfloat16), onehot], axis=0)
    out = jnp.dot(rcombt_ref[...], lhs,
                  preferred_element_type=jnp.float32)

    mu = out[0:1, :] + scal_ref[4]
    hv = out[0:16, :] * mask_ref[...] + biasm_ref[...]
    var = _sublane_sum(hv * hv)
    std = jnp.sqrt(var)

    tv = out[128:256, :]
    z = jnp.where(tv <= x, x - tv, 0.0)
    zs = _sublane_sum(z)
    k0, k256, alf, kp = (scal_ref[0], scal_ref[1], scal_ref[2],
                         scal_ref[3])
    pcs = out[n_out + 1:n_out + 2, :] + out[n_out + 2:n_out + 3, :]
    bstar = out[n_out + 3:n_out + 4, :]
    s = (k0 + (k256 * bstar - alf) * x) + kp * (zs - pcs)
    mu_ref[0] = mu
    std_ref[0] = std * s


def kernel(x, w1, b1, w2, b2, wmu, bmu, wsg, bsg, train_x, alpha):
    B, D = x.shape
    H = w1.shape[0]
    n_out = wmu.shape[0]
    T = train_x.shape[0]

    TM = _TM
    B_pad = ((B + TM - 1) // TM) * TM
    x_p = jnp.pad(x, ((0, B_pad - B), (0, 0))) if B_pad != B else x
    G = B_pad // TM
    xr = x_p.reshape(G, 1, TM)
    grid = (G,)

    w1c = w1.reshape(H, 1)
    b1c = b1.reshape(H, 1)
    w2bf = w2.astype(jnp.bfloat16)
    b2c = b2.reshape(H, 1)

    wmut = wmu.T
    wsgt = wsg.T
    wmu_mean = jnp.mean(wmut, axis=1, keepdims=True)
    csig = wsgt - jnp.mean(wsgt, axis=1, keepdims=True)

    BIG = jnp.float32(1e30)
    nb = max((T + _BW - 1) // _BW, 1)
    Tp = nb * _BW
    ts = jnp.sort(train_x.reshape(-1))
    if Tp != T:
        ts_p = jnp.concatenate([ts, jnp.full((Tp - T,), BIG, jnp.float32)])
    else:
        ts_p = ts
    tmat = ts_p.reshape(nb, _BW)
    bsum = jnp.sum(jnp.where(tmat >= BIG, 0.0, tmat), axis=1)
    pc = jnp.concatenate([jnp.zeros((1,), jnp.float32),
                          jnp.cumsum(bsum)[:-1]])
    pc_hi = pc.astype(jnp.bfloat16).astype(jnp.float32)
    pc_lo = pc - pc_hi
    piv = tmat[:, 0].at[0].set(-BIG)
    pivs = jnp.concatenate([piv, jnp.full((8,), BIG, jnp.float32)])
    s_tot = jnp.sum(ts)

    sig_scale = 1.0 / jnp.sqrt(jnp.float32(n_out - 1))
    rcombt = jnp.zeros((256, H + nb), jnp.float32)
    rcombt = rcombt.at[0:1, :H].set(wmu_mean.T)
    rcombt = rcombt.at[1:1 + n_out, :H].set(csig.T)
    rcombt = rcombt.at[n_out + 1, H:].set(pc_hi)
    rcombt = rcombt.at[n_out + 2, H:].set(pc_lo)
    rcombt = rcombt.at[n_out + 3, H:].set(
        jnp.arange(nb, dtype=jnp.float32))
    rcombt = rcombt.at[128:256, H:].set(tmat.T)
    rcombt_bf = rcombt.astype(jnp.bfloat16)

    biasm_col = jnp.zeros((16, 1), jnp.float32)
    biasm_col = biasm_col.at[1:1 + n_out, 0].set(
        (bsg - jnp.mean(bsg)) * sig_scale)
    mask_col = jnp.zeros((16, 1), jnp.float32)
    mask_col = mask_col.at[1:1 + n_out, 0].set(sig_scale)

    p_col = pivs.reshape(nb + 8, 1)

    alpha_f = jnp.asarray(alpha, jnp.float32).reshape(())
    k2 = alpha_f / jnp.float32(T)
    k0 = 1.0 + k2 * s_tot
    k256 = k2 * jnp.float32(2 * _BW)
    kp = 2.0 * k2
    mub = jnp.mean(bmu)
    scal = jnp.stack([k0, k256, alpha_f, kp, mub])

    row3 = lambda i: (i, 0, 0)
    const = lambda i: (0, 0)
    in_specs = [
        pl.BlockSpec(memory_space=pltpu.MemorySpace.SMEM),
        pl.BlockSpec((1, 1, TM), row3),
        pl.BlockSpec((H, 1), const), pl.BlockSpec((H, 1), const),
        pl.BlockSpec((H, H), const), pl.BlockSpec((H, 1), const),
        pl.BlockSpec((256, H + nb), const),
        pl.BlockSpec((16, 1), const), pl.BlockSpec((16, 1), const),
        pl.BlockSpec((nb + 8, 1), const),
    ]
    out_shape = (jax.ShapeDtypeStruct((G, 1, TM), jnp.float32),
                 jax.ShapeDtypeStruct((G, 1, TM), jnp.float32))
    out_specs = (pl.BlockSpec((1, 1, TM), row3),
                 pl.BlockSpec((1, 1, TM), row3))

    mu, std = pl.pallas_call(
        functools.partial(_fused_kernel, n_out=n_out, nb=nb),
        out_shape=out_shape, grid=grid,
        in_specs=in_specs, out_specs=out_specs,
        compiler_params=pltpu.CompilerParams(
            dimension_semantics=("parallel",)),
    )(scal, xr, w1c, b1c, w2bf, b2c, rcombt_bf, biasm_col, mask_col, p_col)

    return mu.reshape(B_pad, 1)[:B], std.reshape(B_pad, 1)[:B]


# speedup vs baseline: 1.3465x; 1.3465x over previous
import functools

import jax
import jax.numpy as jnp
from jax.experimental import pallas as pl
from jax.experimental.pallas import tpu as pltpu

_BW = 128
_TM = 8192


def _sublane_sum(a):
    s = a.shape[0]
    while s > 1:
        h = s // 2
        a = a[:h] + a[h:s]
        s = h
    return a


def _fused_kernel(scal_ref, x_ref, w1_ref, b1_ref, w2_ref, b2_ref,
                  rcombt_ref, biasm_ref, p_ref,
                  mu_ref, std_ref, *, n_out, nb):
    x = x_ref[0]
    h1 = jnp.maximum(w1_ref[...] * x + b1_ref[...], 0.0)
    h2 = jnp.maximum(
        jnp.dot(w2_ref[...], h1.astype(jnp.bfloat16),
                preferred_element_type=jnp.float32) + b2_ref[...], 0.0)

    pv = p_ref[...]
    cmpb = jnp.where(x >= pv[0:nb], 1.0, 0.0).astype(jnp.bfloat16)

    lhs = jnp.concatenate([h2.astype(jnp.bfloat16), cmpb], axis=0)
    out = jnp.dot(rcombt_ref[...], lhs,
                  preferred_element_type=jnp.float32)

    mu = out[16:17, :] + scal_ref[4]
    hv = out[0:16, :] + biasm_ref[...]
    var = _sublane_sum(hv * hv)
    std = jnp.sqrt(var)

    tv = out[24:152, :]
    zs = _sublane_sum(jnp.maximum(x - tv, 0.0))
    k0, k256, alf, kp = (scal_ref[0], scal_ref[1], scal_ref[2],
                         scal_ref[3])
    pcs = out[17:18, :] + out[18:19, :]
    bstar = out[19:20, :]
    s = (k0 + (k256 * bstar - alf) * x) + kp * (zs - pcs)
    mu_ref[0] = mu
    std_ref[0] = std * s


def kernel(x, w1, b1, w2, b2, wmu, bmu, wsg, bsg, train_x, alpha):
    B, D = x.shape
    H = w1.shape[0]
    n_out = wmu.shape[0]
    T = train_x.shape[0]

    TM = _TM
    B_pad = ((B + TM - 1) // TM) * TM
    x_p = jnp.pad(x, ((0, B_pad - B), (0, 0))) if B_pad != B else x
    G = B_pad // TM
    xr = x_p.reshape(G, 1, TM)
    grid = (G,)

    w1c = w1.reshape(H, 1)
    b1c = b1.reshape(H, 1)
    w2bf = w2.astype(jnp.bfloat16)
    b2c = b2.reshape(H, 1)

    wmut = wmu.T
    wsgt = wsg.T
    wmu_mean = jnp.mean(wmut, axis=1, keepdims=True)
    csig = wsgt - jnp.mean(wsgt, axis=1, keepdims=True)

    BIG = jnp.float32(1e30)
    nb = max((T + _BW - 1) // _BW, 1)
    Tp = nb * _BW
    ts = jnp.sort(train_x.reshape(-1))
    if Tp != T:
        ts_p = jnp.concatenate([ts, jnp.full((Tp - T,), BIG, jnp.float32)])
    else:
        ts_p = ts
    tmat = ts_p.reshape(nb, _BW)
    bsum = jnp.sum(jnp.where(tmat >= BIG, 0.0, tmat), axis=1)
    pc = jnp.concatenate([jnp.zeros((1,), jnp.float32),
                          jnp.cumsum(bsum)[:-1]])
    pc_hi = pc.astype(jnp.bfloat16).astype(jnp.float32)
    pc_lo = pc - pc_hi
    piv = tmat[:, 0].at[0].set(-BIG)
    pivs = jnp.concatenate([piv, jnp.full((8,), BIG, jnp.float32)])
    s_tot = jnp.sum(ts)

    def bdiff(v):
        return jnp.concatenate([v[0:1], v[1:] - v[:-1]], axis=0)

    sig_scale = 1.0 / jnp.sqrt(jnp.float32(n_out - 1))
    pcd = bdiff(pc)
    pcd_hi = pcd.astype(jnp.bfloat16).astype(jnp.float32)
    pcd_lo = pcd - pcd_hi
    rcombt = jnp.zeros((152, H + nb), jnp.float32)
    rcombt = rcombt.at[0:n_out, :H].set(csig.T * sig_scale)
    rcombt = rcombt.at[16:17, :H].set(wmu_mean.T)
    rcombt = rcombt.at[17, H:].set(pcd_hi)
    rcombt = rcombt.at[18, H:].set(pcd_lo)
    rcombt = rcombt.at[19, H:].set(
        bdiff(jnp.arange(nb, dtype=jnp.float32)))
    rcombt = rcombt.at[24:152, H:].set(bdiff(tmat).T)
    rcombt_bf = rcombt.astype(jnp.bfloat16)

    biasm_col = jnp.zeros((16, 1), jnp.float32)
    biasm_col = biasm_col.at[0:n_out, 0].set(
        (bsg - jnp.mean(bsg)) * sig_scale)

    p_col = pivs.reshape(nb + 8, 1)

    alpha_f = jnp.asarray(alpha, jnp.float32).reshape(())
    k2 = alpha_f / jnp.float32(T)
    k0 = 1.0 + k2 * s_tot
    k256 = k2 * jnp.float32(2 * _BW)
    kp = 2.0 * k2
    mub = jnp.mean(bmu)
    scal = jnp.stack([k0, k256, alpha_f, kp, mub])

    row3 = lambda i: (i, 0, 0)
    const = lambda i: (0, 0)
    in_specs = [
        pl.BlockSpec(memory_space=pltpu.MemorySpace.SMEM),
        pl.BlockSpec((1, 1, TM), row3),
        pl.BlockSpec((H, 1), const), pl.BlockSpec((H, 1), const),
        pl.BlockSpec((H, H), const), pl.BlockSpec((H, 1), const),
        pl.BlockSpec((152, H + nb), const),
        pl.BlockSpec((16, 1), const),
        pl.BlockSpec((nb + 8, 1), const),
    ]
    out_shape = (jax.ShapeDtypeStruct((G, 1, TM), jnp.float32),
                 jax.ShapeDtypeStruct((G, 1, TM), jnp.float32))
    out_specs = (pl.BlockSpec((1, 1, TM), row3),
                 pl.BlockSpec((1, 1, TM), row3))

    mu, std = pl.pallas_call(
        functools.partial(_fused_kernel, n_out=n_out, nb=nb),
        out_shape=out_shape, grid=grid,
        in_specs=in_specs, out_specs=out_specs,
        compiler_params=pltpu.CompilerParams(
            dimension_semantics=("parallel",)),
    )(scal, xr, w1c, b1c, w2bf, b2c, rcombt_bf, biasm_col, p_col)

    return mu.reshape(B_pad, 1)[:B], std.reshape(B_pad, 1)[:B]


# speedup vs baseline: 1.7596x; 1.3068x over previous
import functools

import jax
import jax.numpy as jnp
from jax.experimental import pallas as pl
from jax.experimental.pallas import tpu as pltpu

_BW = 128
_TM = 8192


def _sublane_sum(a):
    s = a.shape[0]
    while s > 1:
        h = s // 2
        a = a[:h] + a[h:s]
        s = h
    return a


def _fused_kernel(scal_ref, x_ref, w1_ref, b1_ref, w2_ref, b2_ref,
                  rcombt_ref, biasm_ref, p_ref,
                  mu_ref, std_ref, *, n_out, nb):
    x = x_ref[0]
    xb = x.astype(jnp.bfloat16)
    zero = jnp.bfloat16(0)
    h1 = jnp.maximum(w1_ref[...] * xb + b1_ref[...], zero)
    d2 = jnp.dot(w2_ref[...], h1, preferred_element_type=jnp.float32)
    h2 = jnp.maximum(d2.astype(jnp.bfloat16) + b2_ref[...], zero)

    pv = p_ref[...]
    cmpb = jnp.where(xb >= pv[0:nb], jnp.bfloat16(1), zero)

    lhs = jnp.concatenate([h2, cmpb], axis=0)
    out = jnp.dot(rcombt_ref[...], lhs,
                  preferred_element_type=jnp.float32)

    mu = out[16:17, :] + scal_ref[4]
    hv = out[0:16, :] + biasm_ref[...]
    var = _sublane_sum(hv * hv)
    std = jnp.sqrt(var)

    tv = out[24:152, :]
    zs = _sublane_sum(jnp.maximum(x - tv, 0.0))
    k0, k256, alf, kp = (scal_ref[0], scal_ref[1], scal_ref[2],
                         scal_ref[3])
    pcs = out[17:18, :] + out[18:19, :]
    bstar = out[19:20, :]
    s = (k0 + (k256 * bstar - alf) * x) + kp * (zs - pcs)
    mu_ref[0] = mu
    std_ref[0] = std * s


def kernel(x, w1, b1, w2, b2, wmu, bmu, wsg, bsg, train_x, alpha):
    B, D = x.shape
    H = w1.shape[0]
    n_out = wmu.shape[0]
    T = train_x.shape[0]

    TM = _TM
    B_pad = ((B + TM - 1) // TM) * TM
    x_p = jnp.pad(x, ((0, B_pad - B), (0, 0))) if B_pad != B else x
    G = B_pad // TM
    xr = x_p.reshape(G, 1, TM)
    grid = (G,)

    w1c = w1.reshape(H, 1).astype(jnp.bfloat16)
    b1c = b1.reshape(H, 1).astype(jnp.bfloat16)
    w2bf = w2.astype(jnp.bfloat16)
    b2c = b2.reshape(H, 1).astype(jnp.bfloat16)

    wmut = wmu.T
    wsgt = wsg.T
    wmu_mean = jnp.mean(wmut, axis=1, keepdims=True)
    csig = wsgt - jnp.mean(wsgt, axis=1, keepdims=True)

    BIG = jnp.float32(1e30)
    nb = max((T + _BW - 1) // _BW, 1)
    Tp = nb * _BW
    ts = jnp.sort(train_x.reshape(-1))
    if Tp != T:
        ts_p = jnp.concatenate([ts, jnp.full((Tp - T,), BIG, jnp.float32)])
    else:
        ts_p = ts
    tmat = ts_p.reshape(nb, _BW)
    bsum = jnp.sum(jnp.where(tmat >= BIG, 0.0, tmat), axis=1)
    pc = jnp.concatenate([jnp.zeros((1,), jnp.float32),
                          jnp.cumsum(bsum)[:-1]])
    pc_hi = pc.astype(jnp.bfloat16).astype(jnp.float32)
    pc_lo = pc - pc_hi
    piv = tmat[:, 0].at[0].set(-BIG)
    pivs = jnp.concatenate([piv, jnp.full((8,), BIG, jnp.float32)])
    s_tot = jnp.sum(ts)

    def bdiff(v):
        return jnp.concatenate([v[0:1], v[1:] - v[:-1]], axis=0)

    sig_scale = 1.0 / jnp.sqrt(jnp.float32(n_out - 1))
    pcd = bdiff(pc)
    pcd_hi = pcd.astype(jnp.bfloat16).astype(jnp.float32)
    pcd_lo = pcd - pcd_hi
    rcombt = jnp.zeros((152, H + nb), jnp.float32)
    rcombt = rcombt.at[0:n_out, :H].set(csig.T * sig_scale)
    rcombt = rcombt.at[16:17, :H].set(wmu_mean.T)
    rcombt = rcombt.at[17, H:].set(pcd_hi)
    rcombt = rcombt.at[18, H:].set(pcd_lo)
    rcombt = rcombt.at[19, H:].set(
        bdiff(jnp.arange(nb, dtype=jnp.float32)))
    rcombt = rcombt.at[24:152, H:].set(bdiff(tmat).T)
    rcombt_bf = rcombt.astype(jnp.bfloat16)

    biasm_col = jnp.zeros((16, 1), jnp.float32)
    biasm_col = biasm_col.at[0:n_out, 0].set(
        (bsg - jnp.mean(bsg)) * sig_scale)

    p_col = pivs.reshape(nb + 8, 1).astype(jnp.bfloat16)

    alpha_f = jnp.asarray(alpha, jnp.float32).reshape(())
    k2 = alpha_f / jnp.float32(T)
    k0 = 1.0 + k2 * s_tot
    k256 = k2 * jnp.float32(2 * _BW)
    kp = 2.0 * k2
    mub = jnp.mean(bmu)
    scal = jnp.stack([k0, k256, alpha_f, kp, mub])

    row3 = lambda i: (i, 0, 0)
    const = lambda i: (0, 0)
    in_specs = [
        pl.BlockSpec(memory_space=pltpu.MemorySpace.SMEM),
        pl.BlockSpec((1, 1, TM), row3),
        pl.BlockSpec((H, 1), const), pl.BlockSpec((H, 1), const),
        pl.BlockSpec((H, H), const), pl.BlockSpec((H, 1), const),
        pl.BlockSpec((152, H + nb), const),
        pl.BlockSpec((16, 1), const),
        pl.BlockSpec((nb + 8, 1), const),
    ]
    out_shape = (jax.ShapeDtypeStruct((G, 1, TM), jnp.float32),
                 jax.ShapeDtypeStruct((G, 1, TM), jnp.float32))
    out_specs = (pl.BlockSpec((1, 1, TM), row3),
                 pl.BlockSpec((1, 1, TM), row3))

    mu, std = pl.pallas_call(
        functools.partial(_fused_kernel, n_out=n_out, nb=nb),
        out_shape=out_shape, grid=grid,
        in_specs=in_specs, out_specs=out_specs,
        compiler_params=pltpu.CompilerParams(
            dimension_semantics=("parallel",)),
    )(scal, xr, w1c, b1c, w2bf, b2c, rcombt_bf, biasm_col, p_col)

    return mu.reshape(B_pad, 1)[:B], std.reshape(B_pad, 1)[:B]
